# baseline (speedup 1.0000x reference)
import functools

import jax
import jax.numpy as jnp
from jax import lax
from jax.experimental import pallas as pl
from jax.experimental.pallas import tpu as pltpu
from jax.experimental.pallas import tpu_sc as plsc

F32 = jnp.float32
I32 = jnp.int32

NB = 16
HD = 16
N = 1024
E = 65536
NN = N * N
EBLK = 2048
EW = E // 32
ET = E // 16
ROUNDS = 6
TW = 128
GCH = 512
QROW = NN + 128
_IPAD = 256

_mesh = plsc.VectorSubcoreMesh(core_axis_name="c", subcore_axis_name="s")


def _ln(x, g, b):
    m = jnp.mean(x, axis=-1, keepdims=True)
    v = jnp.mean((x - m) ** 2, axis=-1, keepdims=True)
    return (x - m) / jnp.sqrt(v + 1e-5) * g + b


def _leaky(x):
    return jnp.where(x >= 0, x, 0.1 * x)


def _dot(a, b):
    return jax.lax.dot_general(a, b, (((1,), (0,)), ((), ())),
                               preferred_element_type=F32)


def _eye16():
    return (lax.broadcasted_iota(I32, (NB, NB), 0)
            == lax.broadcasted_iota(I32, (NB, NB), 1)).astype(F32)


def _reduce_kern(x_ref, xo_ref, xi_ref):
    xb = x_ref[0]
    xo_ref[0, 0, :] = jnp.sum(xb, axis=1)
    xi_ref[0, 0, :] = jnp.sum(xb, axis=0)


def _tc_reduce(x):
    xo, xi = pl.pallas_call(
        _reduce_kern,
        grid=(NB,),
        in_specs=[pl.BlockSpec((1, N, N), lambda b: (b, 0, 0))],
        out_specs=[pl.BlockSpec((1, 1, N), lambda b: (b, 0, 0)),
                   pl.BlockSpec((1, 1, N), lambda b: (b, 0, 0))],
        out_shape=[jax.ShapeDtypeStruct((NB, 1, N), F32),
                   jax.ShapeDtypeStruct((NB, 1, N), F32)],
    )(x)
    return xo.reshape(NB, N).T, xi.reshape(NB, N).T


def _tables_kern(x_ref, g_ref, b_ref, wj_ref, wi_ref, wm_ref, t_ref):
    xl = _ln(x_ref[...], g_ref[0], b_ref[0])
    t_ref[:, :HD] = _dot(xl, wj_ref[...])
    t_ref[:, HD:2 * HD] = _dot(xl, wm_ref[...])
    t_ref[:, 2 * HD:3 * HD] = _dot(xl, wi_ref[...])
    t_ref[:, 3 * HD:] = jnp.zeros((N, TW - 3 * HD), F32)


def _tc_tables(xn, g, b, wj, wi, wm):
    nin = xn.shape[1]
    wfull = lambda s: pl.BlockSpec(s, lambda: tuple(0 for _ in s))
    return pl.pallas_call(
        _tables_kern,
        in_specs=[wfull((N, nin)), wfull((1, nin)), wfull((1, nin)),
                  wfull((nin, HD)), wfull((nin, HD)), wfull((nin, HD))],
        out_specs=[wfull((N, TW))],
        out_shape=[jax.ShapeDtypeStruct((N, TW), F32)],
    )(xn, g.reshape(1, -1), b.reshape(1, -1), wj, wi, wm)[0]


def _edge_kern(residual, e_ref, gs_ref, gd_ref, epg_ref, epb_ref, w1_ref,
               b1_ref, w2_ref, b2_ref, pg_ref, pb_ref, fs_ref, fsb_ref,
               fh_ref, fhb_ref, en_ref, msg_ref):
    e = e_ref[...]
    eln = _ln(e, epg_ref[0], epb_ref[0])
    h = _leaky(_dot(eln, w1_ref[...]) + gs_ref[:, :HD]
               + gd_ref[:, 2 * HD:3 * HD] + b1_ref[0])
    d = _dot(h, w2_ref[...]) + b2_ref[0]
    gate = jax.nn.sigmoid(d)
    eo = e + gate * d if residual else gate * d
    eo = _ln(eo, pg_ref[0], pb_ref[0])
    scale = _dot(eo, fs_ref[...]) + fsb_ref[0]
    shift = _dot(eo, fh_ref[...]) + fhb_ref[0]
    msg_ref[:, :HD] = (1.0 + scale) * gs_ref[:, HD:2 * HD] + shift
    msg_ref[:, HD:] = jnp.zeros((e.shape[0], TW - HD), F32)
    en_ref[...] = jnp.maximum(eo, 0.0)


def _tc_edge(e, gs, gd, lp, residual):
    ein = e.shape[1]
    wfull = lambda s: pl.BlockSpec(s, lambda i: tuple(0 for _ in s))
    return pl.pallas_call(
        functools.partial(_edge_kern, residual),
        grid=(E // EBLK,),
        in_specs=[pl.BlockSpec((EBLK, ein), lambda i: (i, 0)),
                  pl.BlockSpec((EBLK, TW), lambda i: (i, 0)),
                  pl.BlockSpec((EBLK, TW), lambda i: (i, 0)),
                  wfull((1, ein)), wfull((1, ein)),
                  wfull((ein, HD)), wfull((1, HD)),
                  wfull((HD, HD)), wfull((1, HD)),
                  wfull((1, HD)), wfull((1, HD)),
                  wfull((HD, HD)), wfull((1, HD)),
                  wfull((HD, HD)), wfull((1, HD))],
        out_specs=[pl.BlockSpec((EBLK, HD), lambda i: (i, 0)),
                   pl.BlockSpec((EBLK, TW), lambda i: (i, 0))],
        out_shape=[jax.ShapeDtypeStruct((E, HD), F32),
                   jax.ShapeDtypeStruct((E, TW), F32)],
    )(e, gs, gd, lp['edge_pre_g'].reshape(1, -1),
      lp['edge_pre_b'].reshape(1, -1), lp['er1_w'][:ein],
      lp['er1_b'].reshape(1, -1), lp['er2_w'], lp['er2_b'].reshape(1, -1),
      lp['edge_post_g'].reshape(1, -1), lp['edge_post_b'].reshape(1, -1),
      lp['fs_w'], lp['fs_b'].reshape(1, -1), lp['fh_w'],
      lp['fh_b'].reshape(1, -1))


def _node_kern(residual, tables, part_ref, xp_ref, f1_ref, f1b_ref, f2_ref,
               f2b_ref, pg_ref, pb_ref, *rest):
    aggr = part_ref[0, :, :HD] + part_ref[1, :, :HD]
    h2 = _leaky(_dot(aggr, f1_ref[...]) + f1b_ref[0])
    xu = _dot(h2, f2_ref[...]) + f2b_ref[0]
    xn = xp_ref[...] + xu if residual else xu
    xn = jnp.maximum(_ln(xn, pg_ref[0], pb_ref[0]), 0.0)
    if tables:
        (ng_ref, nb_ref, wj_ref, wi_ref, wm_ref, x_ref, t_ref) = rest
        x_ref[...] = xn
        xl = _ln(xn, ng_ref[0], nb_ref[0])
        t_ref[:, :HD] = _dot(xl, wj_ref[...])
        t_ref[:, HD:2 * HD] = _dot(xl, wm_ref[...])
        t_ref[:, 2 * HD:3 * HD] = _dot(xl, wi_ref[...])
        t_ref[:, 3 * HD:] = jnp.zeros((N, TW - 3 * HD), F32)
    else:
        (x_ref,) = rest
        x_ref[...] = xn


def _tc_node(part, xprev, lp, nlp, residual):
    wfull = lambda s: pl.BlockSpec(s, lambda: tuple(0 for _ in s))
    tables = nlp is not None
    ins = [part, xprev, lp['ff1_w'], lp['ff1_b'].reshape(1, -1),
           lp['ff2_w'], lp['ff2_b'].reshape(1, -1),
           lp['node_post_g'].reshape(1, -1), lp['node_post_b'].reshape(1, -1)]
    in_specs = [wfull((2, N, TW)), wfull(xprev.shape), wfull((HD, HD)),
                wfull((1, HD)), wfull((HD, HD)), wfull((1, HD)),
                wfull((1, HD)), wfull((1, HD))]
    out_specs = [wfull((N, HD))]
    out_shape = [jax.ShapeDtypeStruct((N, HD), F32)]
    if tables:
        ein = HD
        ins += [nlp['node_pre_g'].reshape(1, -1),
                nlp['node_pre_b'].reshape(1, -1),
                nlp['er1_w'][ein:ein + HD], nlp['er1_w'][ein + HD:],
                nlp['W_msg']]
        in_specs += [wfull((1, HD)), wfull((1, HD)), wfull((HD, HD)),
                     wfull((HD, HD)), wfull((HD, HD))]
        out_specs += [wfull((N, TW))]
        out_shape += [jax.ShapeDtypeStruct((N, TW), F32)]
    return pl.pallas_call(
        functools.partial(_node_kern, residual, tables),
        in_specs=in_specs, out_specs=out_specs, out_shape=out_shape,
    )(*ins)


def _jk_kern(x1_ref, x2_ref, x3_ref, w0_ref, b0_ref, w1_ref, b1_ref,
             w2_ref, b2_ref, t_ref):
    x1, x2, x3 = x1_ref[...], x2_ref[...], x3_ref[...]
    t_ref[:, :HD] = _dot(x1, w0_ref[...]) + b0_ref[0]
    t_ref[:, HD:2 * HD] = (_dot(x1, w1_ref[:HD]) + _dot(x2, w1_ref[HD:])
                           + b1_ref[0])
    t_ref[:, 2 * HD:3 * HD] = (_dot(x1, w2_ref[:HD])
                               + _dot(x2, w2_ref[HD:2 * HD])
                               + _dot(x3, w2_ref[2 * HD:]) + b2_ref[0])
    t_ref[:, 3 * HD:] = jnp.zeros((N, TW - 3 * HD), F32)


def _tc_jk(x1, x2, x3, jk):
    wfull = lambda s: pl.BlockSpec(s, lambda: tuple(0 for _ in s))
    return pl.pallas_call(
        _jk_kern,
        in_specs=[wfull((N, HD)), wfull((N, HD)), wfull((N, HD)),
                  wfull((HD, HD)), wfull((1, HD)),
                  wfull((2 * HD, HD)), wfull((1, HD)),
                  wfull((3 * HD, HD)), wfull((1, HD))],
        out_specs=[wfull((N, TW))],
        out_shape=[jax.ShapeDtypeStruct((N, TW), F32)],
    )(x1, x2, x3, jk[0]['w'], jk[0]['b'].reshape(1, -1),
      jk[1]['w'], jk[1]['b'].reshape(1, -1),
      jk[2]['w'], jk[2]['b'].reshape(1, -1))[0]


def _decode_kern(e1_ref, e2_ref, e3_ref, gs_ref, gd_ref, dg_ref, db_ref,
                 hw_ref, hb_ref, p0_ref, p1_ref, p2_ref):
    eye = _eye16()
    for l, (e_ref, p_ref) in enumerate(((e1_ref, p0_ref), (e2_ref, p1_ref),
                                        (e3_ref, p2_ref))):
        dec = jnp.concatenate(
            [e_ref[...], gs_ref[:, l * HD:(l + 1) * HD],
             gd_ref[:, l * HD:(l + 1) * HD]], axis=-1)
        dec = _ln(dec, dg_ref[0], db_ref[0])
        z = _dot(dec, hw_ref[...]) + hb_ref[0]
        p = jnp.logaddexp(z, 0.0)
        p_ref[...] = jax.lax.dot_general(eye, p, (((1,), (1,)), ((), ())),
                                         preferred_element_type=F32)


def _tc_decode(e1, e2, e3, gjs, gjd, params):
    wfull = lambda s: pl.BlockSpec(s, lambda i: tuple(0 for _ in s))
    eb = lambda w: pl.BlockSpec((EBLK, w), lambda i: (i, 0))
    pt = pl.BlockSpec((NB, EBLK), lambda i: (0, i))
    return pl.pallas_call(
        _decode_kern,
        grid=(E // EBLK,),
        in_specs=[eb(HD), eb(HD), eb(HD), eb(TW), eb(TW),
                  wfull((1, 3 * HD)), wfull((1, 3 * HD)),
                  wfull((3 * HD, HD)), wfull((1, HD))],
        out_specs=[pt, pt, pt],
        out_shape=[jax.ShapeDtypeStruct((NB, E), F32)] * 3,
    )(e1, e2, e3, gjs, gjd, params['dec_ln_g'].reshape(1, -1),
      params['dec_ln_b'].reshape(1, -1), params['head_w'],
      params['head_b'].reshape(1, -1))


_ABLK = 8192


def _assemble_kern(q0_ref, q1_ref, q2_ref, m_ref, o0_ref, o1_ref, o2_ref):
    valid = m_ref[0] > 0
    for q_ref, o_ref in ((q0_ref, o0_ref), (q1_ref, o1_ref),
                         (q2_ref, o2_ref)):
        o_ref[...] = jnp.where(valid, q_ref[...], 0.0)


def _tc_assemble(qt0, qt1, qt2, vmapf):
    qb = pl.BlockSpec((NB, _ABLK), lambda i: (0, i))
    mb = pl.BlockSpec((1, 1, _ABLK), lambda i: (i, 0, 0))
    return pl.pallas_call(
        _assemble_kern,
        grid=(NN // _ABLK,),
        in_specs=[qb, qb, qb, mb],
        out_specs=[qb, qb, qb],
        out_shape=[jax.ShapeDtypeStruct((NB, NN), F32)] * 3,
    )(qt0, qt1, qt2, vmapf)


def _wid():
    return lax.axis_index("s") * 2 + lax.axis_index("c")


@functools.partial(
    pl.kernel, mesh=_mesh,
    out_type=[jax.ShapeDtypeStruct((E, TW), F32),
              jax.ShapeDtypeStruct((E, TW), F32)],
    scratch_types=[pltpu.VMEM((GCH,), I32), pltpu.VMEM((GCH, TW), F32),
                   pltpu.SemaphoreType.DMA],
)
def _sc_gather_pair(t_hbm, src_hbm, dst_hbm, gs_hbm, gd_hbm, iv, gv, sem):
    base = _wid() * EW
    for j in range(EW // GCH):
        off = base + j * GCH
        pltpu.sync_copy(src_hbm.at[pl.ds(off, GCH)], iv)
        pltpu.async_copy(t_hbm.at[iv], gv, sem).wait()
        pltpu.sync_copy(gv, gs_hbm.at[pl.ds(off, GCH)])
        pltpu.sync_copy(dst_hbm.at[pl.ds(off, GCH)], iv)
        pltpu.async_copy(t_hbm.at[iv], gv, sem).wait()
        pltpu.sync_copy(gv, gd_hbm.at[pl.ds(off, GCH)])


@functools.partial(
    pl.kernel, mesh=_mesh,
    out_type=[jax.ShapeDtypeStruct((2, N, TW), F32)],
    scratch_types=[pltpu.VMEM((GCH,), I32), pltpu.VMEM((GCH, TW), F32),
                   pltpu.VMEM_SHARED((N, TW), F32),
                   pltpu.SemaphoreType.DMA],
)
def _sc_scatadd(msg_hbm, dst_hbm, z_hbm, part_hbm, ib, mbuf, shared, sem):
    c = lax.axis_index("c")
    s = lax.axis_index("s")

    @pl.when(s == 0)
    def _():
        pltpu.sync_copy(z_hbm, shared)

    plsc.subcore_barrier()
    base = (s * 2 + c) * EW
    for j in range(EW // GCH):
        off = base + j * GCH
        pltpu.sync_copy(dst_hbm.at[pl.ds(off, GCH)], ib)
        pltpu.sync_copy(msg_hbm.at[pl.ds(off, GCH)], mbuf)
        pltpu.async_copy(mbuf, shared.at[ib], sem, add=True).wait()
    plsc.subcore_barrier()

    @pl.when(s == 0)
    def _():
        pltpu.sync_copy(shared, part_hbm.at[c])


@functools.partial(
    pl.kernel, mesh=_mesh,
    out_type=[jax.ShapeDtypeStruct((NN + _IPAD,), I32),
              jax.ShapeDtypeStruct((E,), I32)],
    scratch_types=[pltpu.VMEM((ET,), I32), pltpu.VMEM((ET,), I32),
                   pltpu.VMEM((ET,), I32), pltpu.VMEM((ET,), I32),
                   pltpu.VMEM((4096,), I32), pltpu.SemaphoreType.DMA],
)
def _sc_rounds(key_hbm, ids_hbm, idmap_hbm, dest_hbm,
               keyv, idv, chkv, destv, zbuf, sem):
    c = lax.axis_index("c")
    s = lax.axis_index("s")

    @pl.when(c == 0)
    def _():
        def zb(i, _):
            zbuf[pl.ds(i * 16, 16)] = jnp.zeros((16,), I32)
            return 0
        lax.fori_loop(0, 256, zb, 0)
        seg = (NN + _IPAD) // 16
        for j in range(16):
            pltpu.sync_copy(zbuf,
                            idmap_hbm.at[pl.ds(s * seg + j * 4096, 4096)])
        pltpu.sync_copy(zbuf.at[pl.ds(0, 16)],
                        idmap_hbm.at[pl.ds(s * seg + 65536, 16)])
        base = s * ET
        pltpu.sync_copy(key_hbm.at[pl.ds(base, ET)], keyv)
        pltpu.sync_copy(ids_hbm.at[pl.ds(base, ET)], idv)
        plsc.subcore_barrier()
        pltpu.async_copy(idv, idmap_hbm.at[keyv], sem).wait()
        plsc.subcore_barrier()
        sink = NN + s * 16

        def mask_body(i, _):
            kk = keyv[pl.ds(i * 16, 16)]
            ii = idv[pl.ds(i * 16, 16)]
            ch = chkv[pl.ds(i * 16, 16)]
            destv[pl.ds(i * 16, 16)] = jnp.where(ch < ii, kk, sink)
            return 0

        for _r in range(ROUNDS):
            pltpu.async_copy(idmap_hbm.at[keyv], chkv, sem).wait()
            plsc.subcore_barrier()
            lax.fori_loop(0, ET // 16, mask_body, 0)
            pltpu.async_copy(idv, idmap_hbm.at[destv], sem).wait()
            plsc.subcore_barrier()
        pltpu.async_copy(idmap_hbm.at[keyv], chkv, sem).wait()

        def final_body(i, _):
            kk = keyv[pl.ds(i * 16, 16)]
            ii = idv[pl.ds(i * 16, 16)]
            ch = chkv[pl.ds(i * 16, 16)]
            destv[pl.ds(i * 16, 16)] = jnp.where(ch == ii, kk, NN + s)
            return 0

        lax.fori_loop(0, ET // 16, final_body, 0)
        pltpu.sync_copy(destv, dest_hbm.at[pl.ds(base, ET)])


@functools.partial(
    pl.kernel, mesh=_mesh,
    out_type=[jax.ShapeDtypeStruct((NB * QROW,), F32)] * 3,
    scratch_types=[pltpu.VMEM((EW,), I32), pltpu.VMEM((EW,), I32),
                   pltpu.VMEM((EW,), F32), pltpu.SemaphoreType.DMA],
)
def _sc_pscat(pt0_hbm, pt1_hbm, pt2_hbm, dest_hbm, q0_hbm, q1_hbm, q2_hbm,
              dv, ib, pbuf, sem):
    base = _wid() * EW
    pltpu.sync_copy(dest_hbm.at[pl.ds(base, EW)], dv)
    for pt_hbm, q_hbm in ((pt0_hbm, q0_hbm), (pt1_hbm, q1_hbm),
                          (pt2_hbm, q2_hbm)):
        for b in range(NB):
            def bb(i, _):
                ib[pl.ds(i * 16, 16)] = dv[pl.ds(i * 16, 16)] + b * QROW
                return 0
            lax.fori_loop(0, EW // 16, bb, 0)
            pltpu.sync_copy(pt_hbm.at[pl.ds(b * E + base, EW)], pbuf)
            pltpu.async_copy(pbuf, q_hbm.at[ib], sem).wait()


def kernel(x, edge_index, edge_attr, tx, rx, params):
    src = edge_index[0]
    dst = edge_index[1]
    lp = params['layers']

    xo, xi = _tc_reduce(x)
    role = jnp.zeros((N, 3), F32).at[tx, 0].set(1.0).at[rx, 1].set(1.0)
    role = role.at[:, 2].set((jnp.sum(role, axis=1) == 0.0).astype(F32))
    x0 = jnp.concatenate([xo, xi, role], axis=-1)

    ein0 = 2 * HD
    tbl = _tc_tables(x0, lp[0]['node_pre_g'], lp[0]['node_pre_b'],
                     lp[0]['er1_w'][ein0:ein0 + 35],
                     lp[0]['er1_w'][ein0 + 35:], lp[0]['W_msg'])
    zflat = jnp.zeros((N, TW), F32)
    es = []
    xs = []
    xprev = x0
    e = edge_attr
    for l in range(3):
        gs, gd = _sc_gather_pair(tbl, src, dst)
        e, msg = _tc_edge(e, gs, gd, lp[l], residual=(l > 0))
        es.append(e)
        (part,) = _sc_scatadd(msg, dst, zflat)
        nxt = _tc_node(part, xprev, lp[l],
                       lp[l + 1] if l < 2 else None, residual=(l > 0))
        if l < 2:
            xprev, tbl = nxt
        else:
            (xprev,) = nxt
        xs.append(xprev)

    tjk = _tc_jk(xs[0], xs[1], xs[2], params['jk'])
    gjs, gjd = _sc_gather_pair(tjk, src, dst)
    pt0, pt1, pt2 = _tc_decode(es[0], es[1], es[2], gjs, gjd, params)

    key = src * N + dst
    ids = jnp.arange(1, E + 1, dtype=I32)
    idmap, dest = _sc_rounds(key, ids)
    q0, q1, q2 = _sc_pscat(pt0.reshape(NB * E), pt1.reshape(NB * E),
                           pt2.reshape(NB * E), dest)
    vmapf = idmap[:NN].reshape(NN // _ABLK, 1, _ABLK)
    o0, o1, o2 = _tc_assemble(q0.reshape(NB, QROW)[:, :NN],
                              q1.reshape(NB, QROW)[:, :NN],
                              q2.reshape(NB, QROW)[:, :NN], vmapf)
    return (o0.reshape(NB, N, N), o1.reshape(NB, N, N),
            o2.reshape(NB, N, N))
